# Initial kernel scaffold; baseline (speedup 1.0000x reference)
import functools

import jax
import jax.numpy as jnp
from jax import lax
from jax.experimental import pallas as pl
from jax.experimental.pallas import tpu as pltpu
from jax.experimental.pallas import tpu_sc as plsc

VOCAB = 49408
HIDDEN = 512
MAX_POS = 16
B = 4096
T = 16

N = B * T
NW = 32
ROWS_PER_W = N // NW
CHUNK = 64
CHUNKS_PER_W = ROWS_PER_W // CHUNK
NCHUNKS = N // CHUNK
LANES = 16
GROUPS = HIDDEN // LANES


def _body(table_hbm, idx_hbm, pos_hbm, out_hbm, idx_v, pos_v, rows0, rows1,
          sem0, sem1):
    wid = lax.axis_index("s") * 2 + lax.axis_index("c")
    cbase = wid * CHUNKS_PER_W

    pltpu.sync_copy(idx_hbm.at[pl.ds(cbase, CHUNKS_PER_W)], idx_v)
    pltpu.sync_copy(pos_hbm, pos_v)

    def start_gather(c, buf, sem):
        pltpu.async_copy(table_hbm.at[idx_v.at[c]], buf, sem)

    def wait_gather(c, buf, sem):
        pltpu.make_async_copy(table_hbm.at[idx_v.at[c]], buf, sem).wait()

    def process(c, buf):
        def g_body(g, carry):
            col = pl.ds(g * LANES, LANES)
            for p in range(MAX_POS):
                pv = pos_v[p, col]
                for j in range(CHUNK // MAX_POS):
                    r = j * MAX_POS + p
                    buf[r, col] = buf[r, col] + pv
            return carry

        lax.fori_loop(0, GROUPS, g_body, 0)
        pltpu.sync_copy(buf, out_hbm.at[pl.ds((cbase + c) * CHUNK, CHUNK)])

    start_gather(0, rows0, sem0)

    def loop_body(i, carry):
        c0 = 2 * i
        c1 = 2 * i + 1
        start_gather(c1, rows1, sem1)
        wait_gather(c0, rows0, sem0)

        @pl.when(i < CHUNKS_PER_W // 2 - 1)
        def _():
            start_gather(c0 + 2, rows0, sem0)

        process(c0, rows0)
        wait_gather(c1, rows1, sem1)
        process(c1, rows1)
        return carry

    lax.fori_loop(0, CHUNKS_PER_W // 2, loop_body, 0)


@jax.jit
def _embed(ids2d, token_table, position_table):
    mesh = plsc.VectorSubcoreMesh(core_axis_name="c", subcore_axis_name="s")
    k = functools.partial(
        pl.kernel,
        out_type=jax.ShapeDtypeStruct((N, HIDDEN), jnp.float32),
        mesh=mesh,
        scratch_types=[
            pltpu.VMEM((CHUNKS_PER_W, CHUNK), jnp.int32),
            pltpu.VMEM((MAX_POS, HIDDEN), jnp.float32),
            pltpu.VMEM((CHUNK, HIDDEN), jnp.float32),
            pltpu.VMEM((CHUNK, HIDDEN), jnp.float32),
            pltpu.SemaphoreType.DMA,
            pltpu.SemaphoreType.DMA,
        ],
    )(_body)
    return k(token_table, ids2d, position_table)


def kernel(input_ids, token_table, position_table):
    ids2d = input_ids.astype(jnp.int32).reshape(NCHUNKS, CHUNK)
    out = _embed(ids2d, token_table, position_table)
    return out.reshape(B, T, HIDDEN)


# speedup vs baseline: 1.3165x; 1.3165x over previous
import functools

import jax
import jax.numpy as jnp
from jax import lax
from jax.experimental import pallas as pl
from jax.experimental.pallas import tpu as pltpu
from jax.experimental.pallas import tpu_sc as plsc

VOCAB = 49408
HIDDEN = 512
MAX_POS = 16
B = 4096
T = 16

N = B * T
NW = 32
ROWS_PER_W = N // NW
CHUNK = 64
CHUNKS_PER_W = ROWS_PER_W // CHUNK
NCHUNKS = N // CHUNK
LANES = 16
GROUPS = HIDDEN // LANES


def _body(table_hbm, idx_hbm, pos_hbm, out_hbm, idx_v, pos_v, rows0, rows1,
          sem0, sem1):
    wid = lax.axis_index("s") * 2 + lax.axis_index("c")
    cbase = wid * CHUNKS_PER_W

    pltpu.sync_copy(idx_hbm.at[pl.ds(cbase, CHUNKS_PER_W)], idx_v)
    pltpu.sync_copy(pos_hbm, pos_v)

    def start_gather(c, buf, sem):
        pltpu.async_copy(table_hbm.at[idx_v.at[c]], buf, sem)

    def wait_gather(c, buf, sem):
        pltpu.make_async_copy(table_hbm.at[idx_v.at[c]], buf, sem).wait()

    def process(c, buf):
        def g_body(g, carry):
            col = pl.ds(g * LANES, LANES)
            for p in range(MAX_POS):
                pv = pos_v[p, col]
                for j in range(CHUNK // MAX_POS):
                    r = j * MAX_POS + p
                    buf[r, col] = buf[r, col] + pv
            return carry

        lax.fori_loop(0, GROUPS, g_body, 0)
        pltpu.sync_copy(buf, out_hbm.at[pl.ds((cbase + c) * CHUNK, CHUNK)])

    start_gather(0, rows0, sem0)

    def loop_body(i, carry):
        c0 = 2 * i
        c1 = 2 * i + 1
        start_gather(c1, rows1, sem1)
        wait_gather(c0, rows0, sem0)
        process(c0, rows0)

        @pl.when(i < CHUNKS_PER_W // 2 - 1)
        def _():
            start_gather(c0 + 2, rows0, sem0)

        wait_gather(c1, rows1, sem1)
        process(c1, rows1)
        return carry

    lax.fori_loop(0, CHUNKS_PER_W // 2, loop_body, 0)


@jax.jit
def _embed(ids2d, token_table, position_table):
    mesh = plsc.VectorSubcoreMesh(core_axis_name="c", subcore_axis_name="s")
    k = functools.partial(
        pl.kernel,
        out_type=jax.ShapeDtypeStruct((N, HIDDEN), jnp.float32),
        mesh=mesh,
        scratch_types=[
            pltpu.VMEM((CHUNKS_PER_W, CHUNK), jnp.int32),
            pltpu.VMEM((MAX_POS, HIDDEN), jnp.float32),
            pltpu.VMEM((CHUNK, HIDDEN), jnp.float32),
            pltpu.VMEM((CHUNK, HIDDEN), jnp.float32),
            pltpu.SemaphoreType.DMA,
            pltpu.SemaphoreType.DMA,
        ],
    )(_body)
    return k(token_table, ids2d, position_table)


def kernel(input_ids, token_table, position_table):
    ids2d = input_ids.astype(jnp.int32).reshape(NCHUNKS, CHUNK)
    out = _embed(ids2d, token_table, position_table)
    return out.reshape(B, T, HIDDEN)
